# baseline (speedup 1.0000x reference)
import functools

import jax
import jax.numpy as jnp
from jax import lax
from jax.experimental import pallas as pl
from jax.experimental.pallas import tpu as pltpu
from jax.experimental.pallas import tpu_sc as plsc

B = 8
C = 3
H = 512
W = 512
NPIX = H * W
NCHUNK = 4
ROWS = H // NCHUNK
K_RANK = int(W * H * 0.4 - 1)
N_TOTAL = B * C * NPIX
SIGN = -2**31

CHUNK = ROWS * W
NB1 = 1024
NB2 = 2048
NB3 = 2048
CCAP = 25600


def _prep_body(yt_ref, yp_ref, key_ref, a_ref):
    yp = yp_ref[0]
    yt = yt_ref[0]
    g = (yp[0] + yp[1] + yp[2]) / 3.0
    bits = lax.bitcast_convert_type(g, jnp.int32)
    ukey = jnp.where(bits < 0, ~bits, bits ^ SIGN)
    key_ref[0, 0] = ukey
    a_ref[0, 0] = (jnp.abs(yp[0] - yt[0]) + jnp.abs(yp[1] - yt[1])
                   + jnp.abs(yp[2] - yt[2]))


def _prep(y_true, y_pred):
    return pl.pallas_call(
        _prep_body,
        grid=(B, NCHUNK),
        in_specs=[
            pl.BlockSpec((1, C, ROWS, W), lambda b, q: (b, 0, q, 0)),
            pl.BlockSpec((1, C, ROWS, W), lambda b, q: (b, 0, q, 0)),
        ],
        out_specs=[
            pl.BlockSpec((1, 1, ROWS, W), lambda b, q: (b, q, 0, 0)),
            pl.BlockSpec((1, 1, ROWS, W), lambda b, q: (b, q, 0, 0)),
        ],
        out_shape=[
            jax.ShapeDtypeStruct((B, NCHUNK, ROWS, W), jnp.int32),
            jax.ShapeDtypeStruct((B, NCHUNK, ROWS, W), jnp.float32),
        ],
    )(y_true, y_pred)


def _select_body(keys_hbm, a_hbm, out_hbm, keys_v, hist_v, red_v, tmp_v,
                 sem_k, sem_a, sem_b, board):
    c = lax.axis_index("c")
    s = lax.axis_index("s")
    img_in_core = lax.shift_right_logical(s, 2)
    q = lax.bitwise_and(s, 3)
    b = c * 4 + img_in_core
    wid = c * 16 + s

    cp = pltpu.async_copy(keys_hbm.at[b, q], keys_v, sem_k)

    lane = lax.iota(jnp.int32, 16)
    ones = jnp.ones((16,), jnp.float32)
    zeros16 = jnp.zeros((16,), jnp.float32)

    def zero_hist(nwords):
        def zr(i, _):
            for u in range(32):
                hist_v[pl.ds(i * 512 + u * 16, 16)] = zeros16
            return 0
        lax.fori_loop(0, nwords // 512, zr, 0)

    def reduce_and_combine(nbins):
        def red(j, _):
            vs = [hist_v[pl.ds(p * nbins + j * 16, 16)] for p in range(16)]
            while len(vs) > 1:
                vs = [vs[a] + vs[a + 1] for a in range(0, len(vs), 2)]
            red_v[pl.ds(j * 16, 16)] = vs[0]
            return 0
        lax.fori_loop(0, nbins // 16, red, 0)
        plsc.subcore_barrier()
        pltpu.sync_copy(red_v.at[pl.ds(0, nbins)], board.at[s, pl.ds(0, nbins)])
        plsc.subcore_barrier()
        base = img_in_core * 4
        pltpu.sync_copy(board.at[base, pl.ds(0, nbins)], red_v.at[pl.ds(0, nbins)])
        for qq in range(1, 4):
            pltpu.sync_copy(board.at[base + qq, pl.ds(0, nbins)],
                            tmp_v.at[pl.ds(0, nbins)])

            def addup(j, _):
                red_v[pl.ds(j * 16, 16)] = (red_v[pl.ds(j * 16, 16)]
                                            + tmp_v[pl.ds(j * 16, 16)])
                return 0
            lax.fori_loop(0, nbins // 16, addup, 0)

    ks_idx = [jnp.maximum(lane - sh, 0) for sh in (1, 2, 4, 8)]
    ks_msk = [lane >= sh for sh in (1, 2, 4, 8)]
    last_l = jnp.full((16,), 15, jnp.int32)

    def _prefix16(v):
        for ix, mk in zip(ks_idx, ks_msk):
            g = v.at[ix].get(mode="promise_in_bounds")
            v = v + jnp.where(mk, g, jnp.zeros_like(g))
        return v

    def find_bin(nbins, k):
        kv = jnp.full((16,), k, jnp.float32)
        zv = jnp.zeros((16,), jnp.float32)

        def fb(j, carry):
            cnt_a, pb_a, run = carry
            incl = _prefix16(red_v[pl.ds(j * 16, 16)]) + run
            le = incl <= kv
            cnt_a = cnt_a + jnp.where(le, jnp.float32(1), jnp.float32(0))
            pb_a = jnp.maximum(pb_a, jnp.where(le, incl, jnp.float32(0)))
            run = incl.at[last_l].get(mode="promise_in_bounds")
            return (cnt_a, pb_a, run)

        cnt_a, pb_a, _ = lax.fori_loop(0, nbins // 16, fb, (zv, zv, zv))
        d = lax.convert_element_type(jnp.sum(cnt_a), jnp.int32)
        pb = jnp.max(pb_a)
        return d, k - pb

    def scan_level(nbins, digit_fn, pmask_fn):
        lp = lane * nbins
        UN = 16

        def body(i, _):
            vs = [keys_v[pl.ds(i * (UN * 16) + u * 16, 16)]
                  for u in range(UN)]
            idxs = [jnp.bitwise_or(digit_fn(v), lp) for v in vs]
            if pmask_fn is None:
                for idx in idxs:
                    plsc.addupdate_scatter(hist_v, [idx], ones)
            else:
                pms = [pmask_fn(v) for v in vs]
                for idx, pm in zip(idxs, pms):
                    plsc.addupdate_scatter(hist_v, [idx], ones, mask=pm)
            return 0
        lax.fori_loop(0, CHUNK // (UN * 16), body, 0)

    zero_hist(NB1 * 16)
    cp.wait()
    scan_level(NB1, lambda v: lax.shift_right_logical(v, 22), None)
    reduce_and_combine(NB1)
    d1, r1 = find_bin(NB1, jnp.float32(K_RANK))

    zero_hist(NB2 * 16)
    d1v = jnp.full((16,), d1, jnp.int32)
    scan_level(NB2,
               lambda v: lax.bitwise_and(lax.shift_right_logical(v, 11),
                                         jnp.int32(NB2 - 1)),
               lambda v: lax.shift_right_logical(v, 22) == d1v)
    reduce_and_combine(NB2)
    d2, r2 = find_bin(NB2, r1)

    zero_hist(NB3 * 16)
    d12v = jnp.full((16,), d1 * NB2 + d2, jnp.int32)
    scan_level(NB3,
               lambda v: lax.bitwise_and(v, jnp.int32(NB3 - 1)),
               lambda v: lax.shift_right_logical(v, 11) == d12v)
    reduce_and_combine(NB3)
    d3, _ = find_bin(NB3, r2)

    t_u = lax.shift_left(d1, 22) + lax.shift_left(d2, 11) + d3
    t_sv = jnp.full((16,), t_u ^ SIGN, jnp.int32)

    PIECE = 16384
    sems = (sem_a, sem_b)
    w_hi = jnp.full((16,), 0.8, jnp.float32)
    w_lo = jnp.full((16,), 0.2, jnp.float32)

    def start_piece(p):
        return pltpu.async_copy(
            a_hbm.at[b, q, pl.ds(p * PIECE, PIECE)],
            hist_v.at[pl.ds((p % 2) * PIECE, PIECE)], sems[p % 2])

    cpa = start_piece(0)
    accs = tuple(jnp.zeros((16,), jnp.float32) for _ in range(16))
    for p in range(CHUNK // PIECE):
        nxt = start_piece(p + 1) if p + 1 < CHUNK // PIECE else None
        cpa.wait()
        hoff = (p % 2) * PIECE
        koff = p * PIECE

        def pb_body(j, accs, hoff=hoff, koff=koff):
            b0 = j * 256
            avs = [hist_v[pl.ds(hoff + b0 + u * 16, 16)] for u in range(16)]
            kvs = [keys_v[pl.ds(koff + b0 + u * 16, 16)] for u in range(16)]
            ws = [jnp.where((kv ^ SIGN) <= t_sv, w_hi, w_lo) for kv in kvs]
            return tuple(acc + av * w
                         for acc, av, w in zip(accs, avs, ws))

        accs = lax.fori_loop(0, PIECE // 256, pb_body, accs)
        cpa = nxt

    tot = accs[0]
    for acc in accs[1:]:
        tot = tot + acc
    for u in range(1, 8):
        red_v[pl.ds(u * 16, 16)] = jnp.zeros((16,), jnp.float32)
    red_v[pl.ds(0, 16)] = tot
    pltpu.sync_copy(red_v.at[pl.ds(0, 128)], out_hbm.at[wid])


@functools.cache
def _select():
    mesh = plsc.VectorSubcoreMesh(core_axis_name="c", subcore_axis_name="s")
    return pl.kernel(
        _select_body,
        out_type=jax.ShapeDtypeStruct((2 * 16, 128), jnp.float32),
        mesh=mesh,
        scratch_types=[
            pltpu.VMEM((CHUNK,), jnp.int32),
            pltpu.VMEM((16 * NB2,), jnp.float32),
            pltpu.VMEM((NB2,), jnp.float32),
            pltpu.VMEM((NB2,), jnp.float32),
            pltpu.SemaphoreType.DMA,
            pltpu.SemaphoreType.DMA,
            pltpu.SemaphoreType.DMA,
            pltpu.VMEM_SHARED((16, NB2), jnp.float32),
        ],
        compiler_params=pltpu.CompilerParams(needs_layout_passes=False),
    )


def kernel(y_true, y_pred):
    keys, a = _prep(y_true, y_pred)
    sums = _select()(keys.reshape(B, NCHUNK, CHUNK),
                     a.reshape(B, NCHUNK, CHUNK))
    return jnp.reshape(jnp.sum(sums) * jnp.float32(1.0 / N_TOTAL), ())


# speedup vs baseline: 1.0030x; 1.0030x over previous
import functools

import jax
import jax.numpy as jnp
from jax import lax
from jax.experimental import pallas as pl
from jax.experimental.pallas import tpu as pltpu
from jax.experimental.pallas import tpu_sc as plsc

B = 8
C = 3
H = 512
W = 512
NPIX = H * W
NCHUNK = 4
ROWS = H // NCHUNK
K_RANK = int(W * H * 0.4 - 1)
N_TOTAL = B * C * NPIX
SIGN = -2**31

CHUNK = ROWS * W
NB1 = 1024
NB2 = 2048
NB3 = 2048
CCAP = 25600


def _prep_body(yt_ref, yp_ref, key_ref, a_ref):
    yp = yp_ref[0]
    yt = yt_ref[0]
    g = (yp[0] + yp[1] + yp[2]) / 3.0
    bits = lax.bitcast_convert_type(g, jnp.int32)
    ukey = jnp.where(bits < 0, ~bits, bits ^ SIGN)
    key_ref[0, 0] = ukey
    a_ref[0, 0] = (jnp.abs(yp[0] - yt[0]) + jnp.abs(yp[1] - yt[1])
                   + jnp.abs(yp[2] - yt[2]))


def _prep(y_true, y_pred):
    return pl.pallas_call(
        _prep_body,
        grid=(B, NCHUNK),
        in_specs=[
            pl.BlockSpec((1, C, ROWS, W), lambda b, q: (b, 0, q, 0)),
            pl.BlockSpec((1, C, ROWS, W), lambda b, q: (b, 0, q, 0)),
        ],
        out_specs=[
            pl.BlockSpec((1, 1, ROWS, W), lambda b, q: (b, q, 0, 0)),
            pl.BlockSpec((1, 1, ROWS, W), lambda b, q: (b, q, 0, 0)),
        ],
        out_shape=[
            jax.ShapeDtypeStruct((B, NCHUNK, ROWS, W), jnp.int32),
            jax.ShapeDtypeStruct((B, NCHUNK, ROWS, W), jnp.float32),
        ],
        compiler_params=pltpu.CompilerParams(
            dimension_semantics=("parallel", "parallel")),
    )(y_true, y_pred)


def _select_body(keys_hbm, a_hbm, out_hbm, keys_v, hist_v, red_v, tmp_v,
                 sem_k, sem_a, sem_b, board):
    c = lax.axis_index("c")
    s = lax.axis_index("s")
    img_in_core = lax.shift_right_logical(s, 2)
    q = lax.bitwise_and(s, 3)
    b = c * 4 + img_in_core
    wid = c * 16 + s

    cp = pltpu.async_copy(keys_hbm.at[b, q], keys_v, sem_k)

    lane = lax.iota(jnp.int32, 16)
    ones = jnp.ones((16,), jnp.float32)
    zeros16 = jnp.zeros((16,), jnp.float32)

    def zero_hist(nwords):
        def zr(i, _):
            for u in range(32):
                hist_v[pl.ds(i * 512 + u * 16, 16)] = zeros16
            return 0
        lax.fori_loop(0, nwords // 512, zr, 0)

    def reduce_and_combine(nbins):
        def red(j, _):
            vs = [hist_v[pl.ds(p * nbins + j * 16, 16)] for p in range(16)]
            while len(vs) > 1:
                vs = [vs[a] + vs[a + 1] for a in range(0, len(vs), 2)]
            red_v[pl.ds(j * 16, 16)] = vs[0]
            return 0
        lax.fori_loop(0, nbins // 16, red, 0)
        plsc.subcore_barrier()
        pltpu.sync_copy(red_v.at[pl.ds(0, nbins)], board.at[s, pl.ds(0, nbins)])
        plsc.subcore_barrier()
        base = img_in_core * 4
        pltpu.sync_copy(board.at[base, pl.ds(0, nbins)], red_v.at[pl.ds(0, nbins)])
        for qq in range(1, 4):
            pltpu.sync_copy(board.at[base + qq, pl.ds(0, nbins)],
                            tmp_v.at[pl.ds(0, nbins)])

            def addup(j, _):
                red_v[pl.ds(j * 16, 16)] = (red_v[pl.ds(j * 16, 16)]
                                            + tmp_v[pl.ds(j * 16, 16)])
                return 0
            lax.fori_loop(0, nbins // 16, addup, 0)

    ks_idx = [jnp.maximum(lane - sh, 0) for sh in (1, 2, 4, 8)]
    ks_msk = [lane >= sh for sh in (1, 2, 4, 8)]
    last_l = jnp.full((16,), 15, jnp.int32)

    def _prefix16(v):
        for ix, mk in zip(ks_idx, ks_msk):
            g = v.at[ix].get(mode="promise_in_bounds")
            v = v + jnp.where(mk, g, jnp.zeros_like(g))
        return v

    def find_bin(nbins, k):
        kv = jnp.full((16,), k, jnp.float32)
        zv = jnp.zeros((16,), jnp.float32)

        def fb(j, carry):
            cnt_a, pb_a, run = carry
            incl = _prefix16(red_v[pl.ds(j * 16, 16)]) + run
            le = incl <= kv
            cnt_a = cnt_a + jnp.where(le, jnp.float32(1), jnp.float32(0))
            pb_a = jnp.maximum(pb_a, jnp.where(le, incl, jnp.float32(0)))
            run = incl.at[last_l].get(mode="promise_in_bounds")
            return (cnt_a, pb_a, run)

        cnt_a, pb_a, _ = lax.fori_loop(0, nbins // 16, fb, (zv, zv, zv))
        d = lax.convert_element_type(jnp.sum(cnt_a), jnp.int32)
        pb = jnp.max(pb_a)
        return d, k - pb

    def scan_level(nbins, digit_fn, pmask_fn):
        lp = lane * nbins
        UN = 16

        def body(i, _):
            vs = [keys_v[pl.ds(i * (UN * 16) + u * 16, 16)]
                  for u in range(UN)]
            idxs = [jnp.bitwise_or(digit_fn(v), lp) for v in vs]
            if pmask_fn is None:
                for idx in idxs:
                    plsc.addupdate_scatter(hist_v, [idx], ones)
            else:
                pms = [pmask_fn(v) for v in vs]
                for idx, pm in zip(idxs, pms):
                    plsc.addupdate_scatter(hist_v, [idx], ones, mask=pm)
            return 0
        lax.fori_loop(0, CHUNK // (UN * 16), body, 0)

    zero_hist(NB1 * 16)
    cp.wait()
    scan_level(NB1, lambda v: lax.shift_right_logical(v, 22), None)
    reduce_and_combine(NB1)
    d1, r1 = find_bin(NB1, jnp.float32(K_RANK))

    zero_hist(NB2 * 16)
    d1v = jnp.full((16,), d1, jnp.int32)
    scan_level(NB2,
               lambda v: lax.bitwise_and(lax.shift_right_logical(v, 11),
                                         jnp.int32(NB2 - 1)),
               lambda v: lax.shift_right_logical(v, 22) == d1v)
    reduce_and_combine(NB2)
    d2, r2 = find_bin(NB2, r1)

    zero_hist(NB3 * 16)
    d12v = jnp.full((16,), d1 * NB2 + d2, jnp.int32)
    scan_level(NB3,
               lambda v: lax.bitwise_and(v, jnp.int32(NB3 - 1)),
               lambda v: lax.shift_right_logical(v, 11) == d12v)
    reduce_and_combine(NB3)
    d3, _ = find_bin(NB3, r2)

    t_u = lax.shift_left(d1, 22) + lax.shift_left(d2, 11) + d3
    t_sv = jnp.full((16,), t_u ^ SIGN, jnp.int32)

    PIECE = 16384
    sems = (sem_a, sem_b)
    w_hi = jnp.full((16,), 0.8, jnp.float32)
    w_lo = jnp.full((16,), 0.2, jnp.float32)

    def start_piece(p):
        return pltpu.async_copy(
            a_hbm.at[b, q, pl.ds(p * PIECE, PIECE)],
            hist_v.at[pl.ds((p % 2) * PIECE, PIECE)], sems[p % 2])

    cpa = start_piece(0)
    accs = tuple(jnp.zeros((16,), jnp.float32) for _ in range(16))
    for p in range(CHUNK // PIECE):
        nxt = start_piece(p + 1) if p + 1 < CHUNK // PIECE else None
        cpa.wait()
        hoff = (p % 2) * PIECE
        koff = p * PIECE

        def pb_body(j, accs, hoff=hoff, koff=koff):
            b0 = j * 256
            avs = [hist_v[pl.ds(hoff + b0 + u * 16, 16)] for u in range(16)]
            kvs = [keys_v[pl.ds(koff + b0 + u * 16, 16)] for u in range(16)]
            ws = [jnp.where((kv ^ SIGN) <= t_sv, w_hi, w_lo) for kv in kvs]
            return tuple(acc + av * w
                         for acc, av, w in zip(accs, avs, ws))

        accs = lax.fori_loop(0, PIECE // 256, pb_body, accs)
        cpa = nxt

    tot = accs[0]
    for acc in accs[1:]:
        tot = tot + acc
    for u in range(1, 8):
        red_v[pl.ds(u * 16, 16)] = jnp.zeros((16,), jnp.float32)
    red_v[pl.ds(0, 16)] = tot
    pltpu.sync_copy(red_v.at[pl.ds(0, 128)], out_hbm.at[wid])


@functools.cache
def _select():
    mesh = plsc.VectorSubcoreMesh(core_axis_name="c", subcore_axis_name="s")
    return pl.kernel(
        _select_body,
        out_type=jax.ShapeDtypeStruct((2 * 16, 128), jnp.float32),
        mesh=mesh,
        scratch_types=[
            pltpu.VMEM((CHUNK,), jnp.int32),
            pltpu.VMEM((16 * NB2,), jnp.float32),
            pltpu.VMEM((NB2,), jnp.float32),
            pltpu.VMEM((NB2,), jnp.float32),
            pltpu.SemaphoreType.DMA,
            pltpu.SemaphoreType.DMA,
            pltpu.SemaphoreType.DMA,
            pltpu.VMEM_SHARED((16, NB2), jnp.float32),
        ],
        compiler_params=pltpu.CompilerParams(needs_layout_passes=False),
    )


def kernel(y_true, y_pred):
    keys, a = _prep(y_true, y_pred)
    sums = _select()(keys.reshape(B, NCHUNK, CHUNK),
                     a.reshape(B, NCHUNK, CHUNK))
    return jnp.reshape(jnp.sum(sums) * jnp.float32(1.0 / N_TOTAL), ())


# speedup vs baseline: 1.2051x; 1.2015x over previous
import functools

import jax
import jax.numpy as jnp
from jax import lax
from jax.experimental import pallas as pl
from jax.experimental.pallas import tpu as pltpu
from jax.experimental.pallas import tpu_sc as plsc

B = 8
C = 3
H = 512
W = 512
NPIX = H * W
NCHUNK = 4
ROWS = H // NCHUNK
K_RANK = int(W * H * 0.4 - 1)
N_TOTAL = B * C * NPIX
SIGN = -2**31

CHUNK = ROWS * W
NB1 = 1024
NB2 = 2048
NB3 = 2048
CCAP = 25600


def _prep_body(yt_ref, yp_ref, key_ref, a_ref):
    yp = yp_ref[0]
    yt = yt_ref[0]
    g = (yp[0] + yp[1] + yp[2]) / 3.0
    bits = lax.bitcast_convert_type(g, jnp.int32)
    ukey = jnp.where(bits < 0, ~bits, bits ^ SIGN)
    key_ref[0, 0] = ukey
    a_ref[0, 0] = (jnp.abs(yp[0] - yt[0]) + jnp.abs(yp[1] - yt[1])
                   + jnp.abs(yp[2] - yt[2]))


def _prep(y_true, y_pred):
    return pl.pallas_call(
        _prep_body,
        grid=(B, NCHUNK),
        in_specs=[
            pl.BlockSpec((1, C, ROWS, W), lambda b, q: (b, 0, q, 0)),
            pl.BlockSpec((1, C, ROWS, W), lambda b, q: (b, 0, q, 0)),
        ],
        out_specs=[
            pl.BlockSpec((1, 1, ROWS, W), lambda b, q: (b, q, 0, 0)),
            pl.BlockSpec((1, 1, ROWS, W), lambda b, q: (b, q, 0, 0)),
        ],
        out_shape=[
            jax.ShapeDtypeStruct((B, NCHUNK, ROWS, W), jnp.int32),
            jax.ShapeDtypeStruct((B, NCHUNK, ROWS, W), jnp.float32),
        ],
        compiler_params=pltpu.CompilerParams(
            dimension_semantics=("parallel", "parallel")),
    )(y_true, y_pred)


def _select_body(keys_hbm, a_hbm, out_hbm, keys_v, hist_v, red_v, tmp_v,
                 asg_a, asg_b, sem_k, sem_a, sem_b, board):
    c = lax.axis_index("c")
    s = lax.axis_index("s")
    img_in_core = lax.shift_right_logical(s, 2)
    q = lax.bitwise_and(s, 3)
    b = c * 4 + img_in_core
    wid = c * 16 + s

    cp = pltpu.async_copy(keys_hbm.at[b, q], keys_v, sem_k)

    lane = lax.iota(jnp.int32, 16)
    ones = jnp.ones((16,), jnp.float32)
    zeros16 = jnp.zeros((16,), jnp.float32)

    def zero_hist(nwords):
        def zr(i, _):
            for u in range(32):
                hist_v[pl.ds(i * 512 + u * 16, 16)] = zeros16
            return 0
        lax.fori_loop(0, nwords // 512, zr, 0)

    def reduce_and_combine(nbins):
        def red(j, _):
            vs = [hist_v[pl.ds(p * nbins + j * 16, 16)] for p in range(16)]
            while len(vs) > 1:
                vs = [vs[a] + vs[a + 1] for a in range(0, len(vs), 2)]
            red_v[pl.ds(j * 16, 16)] = vs[0]
            return 0
        lax.fori_loop(0, nbins // 16, red, 0)
        plsc.subcore_barrier()
        pltpu.sync_copy(red_v.at[pl.ds(0, nbins)], board.at[s, pl.ds(0, nbins)])
        plsc.subcore_barrier()
        base = img_in_core * 4
        pltpu.sync_copy(board.at[base, pl.ds(0, nbins)], red_v.at[pl.ds(0, nbins)])
        for qq in range(1, 4):
            pltpu.sync_copy(board.at[base + qq, pl.ds(0, nbins)],
                            tmp_v.at[pl.ds(0, nbins)])

            def addup(j, _):
                red_v[pl.ds(j * 16, 16)] = (red_v[pl.ds(j * 16, 16)]
                                            + tmp_v[pl.ds(j * 16, 16)])
                return 0
            lax.fori_loop(0, nbins // 16, addup, 0)

    ks_idx = [jnp.maximum(lane - sh, 0) for sh in (1, 2, 4, 8)]
    ks_msk = [lane >= sh for sh in (1, 2, 4, 8)]
    last_l = jnp.full((16,), 15, jnp.int32)

    def _prefix16(v):
        for ix, mk in zip(ks_idx, ks_msk):
            g = v.at[ix].get(mode="promise_in_bounds")
            v = v + jnp.where(mk, g, jnp.zeros_like(g))
        return v

    def find_bin(nbins, k):
        kv = jnp.full((16,), k, jnp.float32)
        zv = jnp.zeros((16,), jnp.float32)

        def fb(j, carry):
            cnt_a, pb_a, run = carry
            incl = _prefix16(red_v[pl.ds(j * 16, 16)]) + run
            le = incl <= kv
            cnt_a = cnt_a + jnp.where(le, jnp.float32(1), jnp.float32(0))
            pb_a = jnp.maximum(pb_a, jnp.where(le, incl, jnp.float32(0)))
            run = incl.at[last_l].get(mode="promise_in_bounds")
            return (cnt_a, pb_a, run)

        cnt_a, pb_a, _ = lax.fori_loop(0, nbins // 16, fb, (zv, zv, zv))
        d = lax.convert_element_type(jnp.sum(cnt_a), jnp.int32)
        pb = jnp.max(pb_a)
        return d, k - pb

    def scan_level(nbins, digit_fn, pmask_fn):
        lp = lane * nbins
        UN = 16

        def body(i, _):
            r = lax.shift_right_logical(i, 1)
            co = lax.shift_left(lax.bitwise_and(i, 1), 8)
            vs = [keys_v[r, pl.ds(co + u * 16, 16)]
                  for u in range(UN)]
            idxs = [jnp.bitwise_or(digit_fn(v), lp) for v in vs]
            if pmask_fn is None:
                for idx in idxs:
                    plsc.addupdate_scatter(hist_v, [idx], ones)
            else:
                pms = [pmask_fn(v) for v in vs]
                for idx, pm in zip(idxs, pms):
                    plsc.addupdate_scatter(hist_v, [idx], ones, mask=pm)
            return 0
        lax.fori_loop(0, CHUNK // (UN * 16), body, 0)

    zero_hist(NB1 * 16)
    cp.wait()
    scan_level(NB1, lambda v: lax.shift_right_logical(v, 22), None)
    reduce_and_combine(NB1)
    d1, r1 = find_bin(NB1, jnp.float32(K_RANK))

    zero_hist(NB2 * 16)
    d1v = jnp.full((16,), d1, jnp.int32)
    scan_level(NB2,
               lambda v: lax.bitwise_and(lax.shift_right_logical(v, 11),
                                         jnp.int32(NB2 - 1)),
               lambda v: lax.shift_right_logical(v, 22) == d1v)
    reduce_and_combine(NB2)
    d2, r2 = find_bin(NB2, r1)

    zero_hist(NB3 * 16)
    d12v = jnp.full((16,), d1 * NB2 + d2, jnp.int32)
    scan_level(NB3,
               lambda v: lax.bitwise_and(v, jnp.int32(NB3 - 1)),
               lambda v: lax.shift_right_logical(v, 11) == d12v)
    reduce_and_combine(NB3)
    d3, _ = find_bin(NB3, r2)

    t_u = lax.shift_left(d1, 22) + lax.shift_left(d2, 11) + d3
    t_sv = jnp.full((16,), t_u ^ SIGN, jnp.int32)

    PROWS = 16
    NPIECE = ROWS // PROWS
    asgs = (asg_a, asg_b)
    sems = (sem_a, sem_b)
    w_hi = jnp.full((16,), 0.8, jnp.float32)
    w_lo = jnp.full((16,), 0.2, jnp.float32)

    def start_piece(p):
        return pltpu.async_copy(
            a_hbm.at[b, q, pl.ds(p * PROWS, PROWS)], asgs[p % 2], sems[p % 2])

    cpa = start_piece(0)
    accs = tuple(jnp.zeros((16,), jnp.float32) for _ in range(16))
    for p in range(NPIECE):
        nxt = start_piece(p + 1) if p + 1 < NPIECE else None
        cpa.wait()
        asg = asgs[p % 2]

        def pb_body(j, accs, asg=asg, p=p):
            rr = lax.shift_right_logical(j, 1)
            co = lax.shift_left(lax.bitwise_and(j, 1), 8)
            avs = [asg[rr, pl.ds(co + u * 16, 16)] for u in range(16)]
            kvs = [keys_v[p * PROWS + rr, pl.ds(co + u * 16, 16)]
                   for u in range(16)]
            ws = [jnp.where((kv ^ SIGN) <= t_sv, w_hi, w_lo) for kv in kvs]
            return tuple(acc + av * w
                         for acc, av, w in zip(accs, avs, ws))

        accs = lax.fori_loop(0, PROWS * W // 256, pb_body, accs)
        cpa = nxt

    tot = accs[0]
    for acc in accs[1:]:
        tot = tot + acc
    for u in range(1, 8):
        red_v[pl.ds(u * 16, 16)] = jnp.zeros((16,), jnp.float32)
    red_v[pl.ds(0, 16)] = tot
    pltpu.sync_copy(red_v.at[pl.ds(0, 128)], out_hbm.at[wid])


@functools.cache
def _select():
    mesh = plsc.VectorSubcoreMesh(core_axis_name="c", subcore_axis_name="s")
    return pl.kernel(
        _select_body,
        out_type=jax.ShapeDtypeStruct((2 * 16, 128), jnp.float32),
        mesh=mesh,
        scratch_types=[
            pltpu.VMEM((ROWS, W), jnp.int32),
            pltpu.VMEM((16 * NB2,), jnp.float32),
            pltpu.VMEM((NB2,), jnp.float32),
            pltpu.VMEM((NB2,), jnp.float32),
            pltpu.VMEM((16, W), jnp.float32),
            pltpu.VMEM((16, W), jnp.float32),
            pltpu.SemaphoreType.DMA,
            pltpu.SemaphoreType.DMA,
            pltpu.SemaphoreType.DMA,
            pltpu.VMEM_SHARED((16, NB2), jnp.float32),
        ],
        compiler_params=pltpu.CompilerParams(needs_layout_passes=False),
    )


def kernel(y_true, y_pred):
    keys, a = _prep(y_true, y_pred)
    sums = _select()(keys, a)
    return jnp.reshape(jnp.sum(sums) * jnp.float32(1.0 / N_TOTAL), ())
